# Initial kernel scaffold; baseline (speedup 1.0000x reference)
import jax
import jax.numpy as jnp
from jax.experimental import pallas as pl


def kernel(x, edge_index, W, b):
    raise NotImplementedError("write your pallas kernel here")


# speedup vs baseline: 15.9866x; 15.9866x over previous
import functools

import jax
import jax.numpy as jnp
from jax import lax
from jax.experimental import pallas as pl
from jax.experimental.pallas import tpu as pltpu
from jax.experimental.pallas import tpu_sc as plsc

N = 10000
D = 128
NC = 2
NS = 16
NW = NC * NS

N_PAD = 10240
NODE_CHUNK = N_PAD // NS
TRASH = N
EB = 128

_mesh = plsc.VectorSubcoreMesh(core_axis_name="c", subcore_axis_name="s")


def _deg_body(nchunks, src_h, tgt_h, z1_h, out_h, srcb, tgtb, wb, tgt2d, deg_sh):
    c = lax.axis_index("c")
    s = lax.axis_index("s")
    wid = c * NS + s
    epw = nchunks * EB
    base = wid * epw
    pltpu.sync_copy(z1_h.at[pl.ds(s * NODE_CHUNK, NODE_CHUNK)],
                    deg_sh.at[pl.ds(s * NODE_CHUNK, NODE_CHUNK)])
    pltpu.sync_copy(src_h.at[pl.ds(base, epw)], srcb)
    pltpu.sync_copy(tgt_h.at[pl.ds(base, epw)], tgtb)

    def compute(j, carry):
        for g in range(EB // 16):
            off = j * EB + g * 16
            sv = srcb[pl.ds(off, 16)]
            tv = tgtb[pl.ds(off, 16)]
            wb[pl.ds(off, 16)] = jnp.where(sv != tv, 1.0, 0.0).astype(jnp.float32)
            tgt2d[j, pl.ds(g * 16, 16)] = tv
        return carry

    lax.fori_loop(0, nchunks, compute, 0)
    plsc.subcore_barrier()

    def scat(j, carry):
        pltpu.sync_copy(wb.at[pl.ds(j * EB, EB)],
                        deg_sh.at[tgt2d.at[j]], add=True)
        return carry

    lax.fori_loop(0, nchunks, scat, 0)
    plsc.subcore_barrier()
    pltpu.sync_copy(deg_sh.at[pl.ds(s * NODE_CHUNK, NODE_CHUNK)],
                    out_h.at[c, pl.ds(s * NODE_CHUNK, NODE_CHUNK)])


def _msg_body(nchunks, src_h, tgt_h, h2_h, z2_h, out_h,
              srcb, tgtb, tgt2d, rows, sem, acc_sh):
    c = lax.axis_index("c")
    s = lax.axis_index("s")
    wid = c * NS + s
    epw = nchunks * EB
    base = wid * epw
    pltpu.sync_copy(z2_h, acc_sh.at[pl.ds(s * NODE_CHUNK, NODE_CHUNK)])
    pltpu.sync_copy(src_h.at[pl.ds(base, epw)], srcb)
    pltpu.sync_copy(tgt_h.at[pl.ds(base, epw)], tgtb)

    def compute(j, carry):
        for g in range(EB // 16):
            off = j * EB + g * 16
            sv = srcb[pl.ds(off, 16)]
            tv = tgtb[pl.ds(off, 16)]
            tgt2d[j, pl.ds(g * 16, 16)] = jnp.where(sv == tv, TRASH, tv)
        return carry

    lax.fori_loop(0, nchunks, compute, 0)
    plsc.subcore_barrier()

    def edge_chunk(j, carry):
        pltpu.async_copy(h2_h.at[srcb.at[pl.ds(j * EB, EB)]], rows, sem).wait()
        pltpu.sync_copy(rows, acc_sh.at[tgt2d.at[j]], add=True)
        return carry

    lax.fori_loop(0, nchunks, edge_chunk, 0)
    plsc.subcore_barrier()
    pltpu.sync_copy(acc_sh.at[pl.ds(s * NODE_CHUNK, NODE_CHUNK)],
                    out_h.at[c, pl.ds(s * NODE_CHUNK, NODE_CHUNK)])


def _make_deg(nchunks):
    epw = nchunks * EB
    return pl.kernel(
        functools.partial(_deg_body, nchunks),
        out_type=jax.ShapeDtypeStruct((NC, N_PAD), jnp.float32),
        mesh=_mesh,
        scratch_types=[
            pltpu.VMEM((epw,), jnp.int32),
            pltpu.VMEM((epw,), jnp.int32),
            pltpu.VMEM((epw,), jnp.float32),
            pltpu.VMEM((nchunks, EB), jnp.int32),
            pltpu.VMEM_SHARED((N_PAD,), jnp.float32),
        ],
    )


def _make_msg(nchunks):
    epw = nchunks * EB
    return pl.kernel(
        functools.partial(_msg_body, nchunks),
        out_type=jax.ShapeDtypeStruct((NC, N_PAD, D), jnp.float32),
        mesh=_mesh,
        scratch_types=[
            pltpu.VMEM((epw,), jnp.int32),
            pltpu.VMEM((epw,), jnp.int32),
            pltpu.VMEM((nchunks, EB), jnp.int32),
            pltpu.VMEM((EB, D), jnp.float32),
            pltpu.SemaphoreType.DMA,
            pltpu.VMEM_SHARED((N_PAD, D), jnp.float32),
        ],
    )


def _linear_body(x_ref, p0_ref, p1_ref, w_ref, b_ref, h2_ref, base_ref, dis_ref):
    h = jnp.dot(x_ref[...], w_ref[...], preferred_element_type=jnp.float32)
    deg = 1.0 + p0_ref[...] + p1_ref[...]
    dis = lax.rsqrt(deg)
    h2_ref[...] = dis * h
    base_ref[...] = h * (1.0 / deg) + b_ref[...]
    dis_ref[...] = dis


def _final_body(q0_ref, q1_ref, base_ref, dis_ref, out_ref):
    out_ref[...] = dis_ref[...] * (q0_ref[...] + q1_ref[...]) + base_ref[...]


_ROWS_BLK = 400
_GRID = N // _ROWS_BLK

_linear_call = pl.pallas_call(
    _linear_body,
    grid=(_GRID,),
    in_specs=[
        pl.BlockSpec((_ROWS_BLK, D), lambda i: (i, 0)),
        pl.BlockSpec((_ROWS_BLK, 1), lambda i: (i, 0)),
        pl.BlockSpec((_ROWS_BLK, 1), lambda i: (i, 0)),
        pl.BlockSpec((D, D), lambda i: (0, 0)),
        pl.BlockSpec((1, D), lambda i: (0, 0)),
    ],
    out_specs=[
        pl.BlockSpec((_ROWS_BLK, D), lambda i: (i, 0)),
        pl.BlockSpec((_ROWS_BLK, D), lambda i: (i, 0)),
        pl.BlockSpec((_ROWS_BLK, 1), lambda i: (i, 0)),
    ],
    out_shape=[
        jax.ShapeDtypeStruct((N, D), jnp.float32),
        jax.ShapeDtypeStruct((N, D), jnp.float32),
        jax.ShapeDtypeStruct((N, 1), jnp.float32),
    ],
)

_final_call = pl.pallas_call(
    _final_body,
    grid=(_GRID,),
    in_specs=[
        pl.BlockSpec((_ROWS_BLK, D), lambda i: (i, 0)),
        pl.BlockSpec((_ROWS_BLK, D), lambda i: (i, 0)),
        pl.BlockSpec((_ROWS_BLK, D), lambda i: (i, 0)),
        pl.BlockSpec((_ROWS_BLK, 1), lambda i: (i, 0)),
    ],
    out_specs=pl.BlockSpec((_ROWS_BLK, D), lambda i: (i, 0)),
    out_shape=jax.ShapeDtypeStruct((N, D), jnp.float32),
)


def kernel(x, edge_index, W, b):
    e = edge_index.shape[1]
    nchunks = -(-e // (NW * EB))
    e_pad = nchunks * NW * EB
    src = jnp.zeros((e_pad,), jnp.int32).at[:e].set(edge_index[0].astype(jnp.int32))
    tgt = jnp.zeros((e_pad,), jnp.int32).at[:e].set(edge_index[1].astype(jnp.int32))

    z1 = jnp.zeros((N_PAD,), jnp.float32)
    z2 = jnp.zeros((NODE_CHUNK, D), jnp.float32)

    deg_parts = _make_deg(nchunks)(src, tgt, z1)
    p0 = deg_parts[0, :N, None]
    p1 = deg_parts[1, :N, None]

    h2, base, dis = _linear_call(x, p0, p1, W, b.reshape(1, D))

    acc_parts = _make_msg(nchunks)(src, tgt, h2, z2)
    out = _final_call(acc_parts[0, :N, :], acc_parts[1, :N, :], base, dis)
    return out
